# baseline (speedup 1.0000x reference)
import functools

import jax
import jax.numpy as jnp
from jax import lax
from jax.experimental import pallas as pl
from jax.experimental.pallas import tpu as pltpu
from jax.experimental.pallas import tpu_sc as plsc

N = 10000
B = 4
C = 128
E = 320000

NC = 2
NS = 16
CH = 128

NJ = 10240
SROWS = NJ // NS
ZB = 160

CHUNKS_A = 80
E_PAD_A = NC * NS * CHUNKS_A * CH
NGA = CHUNKS_A // 4

CHUNKS_C = 160
E_PAD_C = NS * CHUNKS_C * CH

_mesh = functools.partial(
    plsc.VectorSubcoreMesh, core_axis_name="c", subcore_axis_name="s"
)


def _deg_body(colp, ones, zeros, deg_out, ones_v, ci_v,
              s0, s1, s2, s3, acc_sh):
  c = lax.axis_index("c")
  s = lax.axis_index("s")
  ss = (s0, s1, s2, s3)
  pltpu.sync_copy(ones, ones_v)
  pltpu.sync_copy(colp.at[pl.ds((c * NS + s) * CHUNKS_A, CHUNKS_A)], ci_v)
  for j in range(SROWS // ZB):
    pltpu.sync_copy(zeros, acc_sh.at[pl.ds(s * SROWS + j * ZB, ZB)])
  plsc.subcore_barrier()

  for j in range(4):
    pltpu.async_copy(ones_v, acc_sh.at[ci_v.at[j]], ss[j], add=True)

  def grp(h, _):
    for j in range(4):
      pltpu.make_async_copy(ones, ones_v, ss[j]).wait()
      pltpu.async_copy(
          ones_v, acc_sh.at[ci_v.at[4 * (h + 1) + j]], ss[j], add=True)
    return 0

  lax.fori_loop(0, NGA - 1, grp, 0)
  for j in range(4):
    pltpu.make_async_copy(ones, ones_v, ss[j]).wait()
  plsc.subcore_barrier()
  pltpu.sync_copy(
      acc_sh.at[pl.ds(s * SROWS, SROWS)],
      deg_out.at[pl.ds(c * NJ + s * SROWS, SROWS)],
  )


def _scat_body(rowsp, colp, xs, out, ri_v, ci_v, gbuf, sem, acc_sh):
  c = lax.axis_index("c")
  s = lax.axis_index("s")
  for r in range(2):
    b = 2 * r + c
    pltpu.sync_copy(
        xs.at[pl.ds(b * NJ + s * SROWS, SROWS)],
        acc_sh.at[pl.ds(s * SROWS, SROWS)],
    )
    plsc.subcore_barrier()
    ebase = b * E_PAD_C + s * (CHUNKS_C * CH)
    cbase = s * (CHUNKS_C * CH)

    def body(g, _):
      pltpu.sync_copy(rowsp.at[pl.ds(ebase + g * CH, CH)], ri_v)
      pltpu.async_copy(xs.at[ri_v], gbuf, sem).wait()
      pltpu.sync_copy(colp.at[pl.ds(cbase + g * CH, CH)], ci_v)
      pltpu.sync_copy(gbuf, acc_sh.at[ci_v], add=True)
      return 0

    lax.fori_loop(0, CHUNKS_C, body, 0)
    plsc.subcore_barrier()
    pltpu.sync_copy(
        acc_sh.at[pl.ds(s * SROWS, SROWS)],
        out.at[pl.ds(b * NJ + s * SROWS, SROWS)],
    )
    plsc.subcore_barrier()


def _dis_block(d0_ref, d1_ref):
  deg = d0_ref[:, 0:1] + d1_ref[:, 0:1] + 1.0
  return lax.rsqrt(deg)


def _mm_body(x_ref, w_ref, d0_ref, d1_ref, o_ref):
  xw = jnp.dot(x_ref[...], w_ref[...], preferred_element_type=jnp.float32)
  o_ref[...] = xw * _dis_block(d0_ref, d1_ref)


def _fin_body(a_ref, d0_ref, d1_ref, b_ref, o_ref):
  o_ref[...] = a_ref[...] * _dis_block(d0_ref, d1_ref) + b_ref[...]


def kernel(x, edge_index, W, b):
  row = edge_index[0].astype(jnp.int32)
  col = edge_index[1].astype(jnp.int32)

  colp_a = jnp.concatenate([col, jnp.full((E_PAD_A - E,), N, jnp.int32)])
  colp_a = colp_a.reshape(NC * NS * CHUNKS_A, CH)
  colp_c = jnp.concatenate([col, jnp.full((E_PAD_C - E,), N, jnp.int32)])
  rowp = jnp.concatenate([row, jnp.zeros((E_PAD_C - E,), jnp.int32)])
  rows4 = (rowp[None, :] + (jnp.arange(B, dtype=jnp.int32) * NJ)[:, None])
  rows4 = rows4.reshape(-1)

  xp = jnp.pad(x, ((0, 0), (0, NJ - N), (0, 0))).reshape(B * NJ, C)

  ones = jnp.ones((CH, 128), jnp.float32)
  zeros = jnp.zeros((ZB, 128), jnp.float32)

  deg_part = pl.kernel(
      _deg_body,
      out_type=jax.ShapeDtypeStruct((NC * NJ, 128), jnp.float32),
      mesh=_mesh(),
      scratch_types=[
          pltpu.VMEM((CH, 128), jnp.float32),
          pltpu.VMEM((CHUNKS_A, CH), jnp.int32),
          pltpu.SemaphoreType.DMA,
          pltpu.SemaphoreType.DMA,
          pltpu.SemaphoreType.DMA,
          pltpu.SemaphoreType.DMA,
          pltpu.VMEM_SHARED((NJ, 128), jnp.float32),
      ],
      name="gcn_degree_sc",
  )(colp_a, ones, zeros)
  deg0 = deg_part[:NJ]
  deg1 = deg_part[NJ:]

  grid = (B * NJ // SROWS,)
  dmap = lambda i: (i % NS, 0)
  xs = pl.pallas_call(
      _mm_body,
      grid=grid,
      in_specs=[
          pl.BlockSpec((SROWS, C), lambda i: (i, 0)),
          pl.BlockSpec((C, C), lambda i: (0, 0)),
          pl.BlockSpec((SROWS, 128), dmap),
          pl.BlockSpec((SROWS, 128), dmap),
      ],
      out_specs=pl.BlockSpec((SROWS, C), lambda i: (i, 0)),
      out_shape=jax.ShapeDtypeStruct((B * NJ, C), jnp.float32),
      name="gcn_xw_scale_tc",
  )(xp, W, deg0, deg1)

  acc = pl.kernel(
      _scat_body,
      out_type=jax.ShapeDtypeStruct((B * NJ, C), jnp.float32),
      mesh=_mesh(),
      scratch_types=[
          pltpu.VMEM((CH,), jnp.int32),
          pltpu.VMEM((CH,), jnp.int32),
          pltpu.VMEM((CH, C), jnp.float32),
          pltpu.SemaphoreType.DMA,
          pltpu.VMEM_SHARED((NJ, C), jnp.float32),
      ],
      name="gcn_edge_scatter_sc",
  )(rows4, colp_c, xs)

  out = pl.pallas_call(
      _fin_body,
      grid=grid,
      in_specs=[
          pl.BlockSpec((SROWS, C), lambda i: (i, 0)),
          pl.BlockSpec((SROWS, 128), dmap),
          pl.BlockSpec((SROWS, 128), dmap),
          pl.BlockSpec((1, C), lambda i: (0, 0)),
      ],
      out_specs=pl.BlockSpec((SROWS, C), lambda i: (i, 0)),
      out_shape=jax.ShapeDtypeStruct((B * NJ, C), jnp.float32),
      name="gcn_finalize_tc",
  )(acc, deg0, deg1, b.reshape(1, C))

  return out.reshape(B, NJ, C)[:, :N]


# speedup vs baseline: 1.0302x; 1.0302x over previous
import functools

import jax
import jax.numpy as jnp
from jax import lax
from jax.experimental import pallas as pl
from jax.experimental.pallas import tpu as pltpu
from jax.experimental.pallas import tpu_sc as plsc

N = 10000
B = 4
C = 128
E = 320000

NC = 2
NS = 16
CH = 128

NJ = 10240
SROWS = NJ // NS
ZB = 160

CHUNKS_A = 80
E_PAD_A = NC * NS * CHUNKS_A * CH
NGA = CHUNKS_A // 4

CHUNKS_C = 160
E_PAD_C = NS * CHUNKS_C * CH

_mesh = functools.partial(
    plsc.VectorSubcoreMesh, core_axis_name="c", subcore_axis_name="s"
)


def _deg_body(colp, ones, zeros, deg_out, ones_v, ci_v, acc_sh):
  c = lax.axis_index("c")
  s = lax.axis_index("s")
  pltpu.sync_copy(ones, ones_v)
  for j in range(SROWS // ZB):
    pltpu.sync_copy(zeros, acc_sh.at[pl.ds(s * SROWS + j * ZB, ZB)])
  plsc.subcore_barrier()

  base = (c * NS + s) * (CHUNKS_A * CH)

  def grp(g, _):
    pltpu.sync_copy(colp.at[pl.ds(base + g * CH, CH)], ci_v)
    pltpu.sync_copy(ones_v, acc_sh.at[ci_v], add=True)
    return 0

  lax.fori_loop(0, CHUNKS_A, grp, 0)
  plsc.subcore_barrier()
  pltpu.sync_copy(
      acc_sh.at[pl.ds(s * SROWS, SROWS)],
      deg_out.at[pl.ds(c * NJ + s * SROWS, SROWS)],
  )


def _scat_body(rowsp, colp, xs, out, ri_v, ci_v, gbuf, sem, acc_sh):
  c = lax.axis_index("c")
  s = lax.axis_index("s")
  for r in range(2):
    b = 2 * r + c
    pltpu.sync_copy(
        xs.at[pl.ds(b * NJ + s * SROWS, SROWS)],
        acc_sh.at[pl.ds(s * SROWS, SROWS)],
    )
    plsc.subcore_barrier()
    ebase = b * E_PAD_C + s * (CHUNKS_C * CH)
    cbase = s * (CHUNKS_C * CH)

    def body(g, _):
      pltpu.sync_copy(rowsp.at[pl.ds(ebase + g * CH, CH)], ri_v)
      pltpu.async_copy(xs.at[ri_v], gbuf, sem).wait()
      pltpu.sync_copy(colp.at[pl.ds(cbase + g * CH, CH)], ci_v)
      pltpu.sync_copy(gbuf, acc_sh.at[ci_v], add=True)
      return 0

    lax.fori_loop(0, CHUNKS_C, body, 0)
    plsc.subcore_barrier()
    pltpu.sync_copy(
        acc_sh.at[pl.ds(s * SROWS, SROWS)],
        out.at[pl.ds(b * NJ + s * SROWS, SROWS)],
    )
    plsc.subcore_barrier()


def _dis_block(d0_ref, d1_ref):
  deg = d0_ref[:, 0:1] + d1_ref[:, 0:1] + 1.0
  return lax.rsqrt(deg)


def _mm_body(x_ref, w_ref, d0_ref, d1_ref, o_ref):
  xw = jnp.dot(x_ref[...], w_ref[...], preferred_element_type=jnp.float32)
  o_ref[...] = xw * _dis_block(d0_ref, d1_ref)


def _fin_body(a_ref, d0_ref, d1_ref, b_ref, o_ref):
  o_ref[...] = a_ref[...] * _dis_block(d0_ref, d1_ref) + b_ref[...]


def kernel(x, edge_index, W, b):
  row = edge_index[0].astype(jnp.int32)
  col = edge_index[1].astype(jnp.int32)

  colp_a = jnp.concatenate([col, jnp.full((E_PAD_A - E,), N, jnp.int32)])
  colp_c = jnp.concatenate([col, jnp.full((E_PAD_C - E,), N, jnp.int32)])
  rowp = jnp.concatenate([row, jnp.zeros((E_PAD_C - E,), jnp.int32)])
  rows4 = (rowp[None, :] + (jnp.arange(B, dtype=jnp.int32) * NJ)[:, None])
  rows4 = rows4.reshape(-1)

  xp = jnp.pad(x, ((0, 0), (0, NJ - N), (0, 0))).reshape(B * NJ, C)

  ones = jnp.ones((CH, 128), jnp.float32)
  zeros = jnp.zeros((ZB, 128), jnp.float32)

  deg_part = pl.kernel(
      _deg_body,
      out_type=jax.ShapeDtypeStruct((NC * NJ, 128), jnp.float32),
      mesh=_mesh(),
      scratch_types=[
          pltpu.VMEM((CH, 128), jnp.float32),
          pltpu.VMEM((CH,), jnp.int32),
          pltpu.VMEM_SHARED((NJ, 128), jnp.float32),
      ],
      name="gcn_degree_sc",
  )(colp_a, ones, zeros)
  deg0 = deg_part[:NJ]
  deg1 = deg_part[NJ:]

  grid = (B * NJ // SROWS,)
  dmap = lambda i: (i % NS, 0)
  xs = pl.pallas_call(
      _mm_body,
      grid=grid,
      in_specs=[
          pl.BlockSpec((SROWS, C), lambda i: (i, 0)),
          pl.BlockSpec((C, C), lambda i: (0, 0)),
          pl.BlockSpec((SROWS, 128), dmap),
          pl.BlockSpec((SROWS, 128), dmap),
      ],
      out_specs=pl.BlockSpec((SROWS, C), lambda i: (i, 0)),
      out_shape=jax.ShapeDtypeStruct((B * NJ, C), jnp.float32),
      name="gcn_xw_scale_tc",
  )(xp, W, deg0, deg1)

  acc = pl.kernel(
      _scat_body,
      out_type=jax.ShapeDtypeStruct((B * NJ, C), jnp.float32),
      mesh=_mesh(),
      scratch_types=[
          pltpu.VMEM((CH,), jnp.int32),
          pltpu.VMEM((CH,), jnp.int32),
          pltpu.VMEM((CH, C), jnp.float32),
          pltpu.SemaphoreType.DMA,
          pltpu.VMEM_SHARED((NJ, C), jnp.float32),
      ],
      name="gcn_edge_scatter_sc",
  )(rows4, colp_c, xs)

  out = pl.pallas_call(
      _fin_body,
      grid=grid,
      in_specs=[
          pl.BlockSpec((SROWS, C), lambda i: (i, 0)),
          pl.BlockSpec((SROWS, 128), dmap),
          pl.BlockSpec((SROWS, 128), dmap),
          pl.BlockSpec((1, C), lambda i: (0, 0)),
      ],
      out_specs=pl.BlockSpec((SROWS, C), lambda i: (i, 0)),
      out_shape=jax.ShapeDtypeStruct((B * NJ, C), jnp.float32),
      name="gcn_finalize_tc",
  )(acc, deg0, deg1, b.reshape(1, C))

  return out.reshape(B, NJ, C)[:, :N]


# speedup vs baseline: 1.5645x; 1.5186x over previous
import functools

import jax
import jax.numpy as jnp
from jax import lax
from jax.experimental import pallas as pl
from jax.experimental.pallas import tpu as pltpu
from jax.experimental.pallas import tpu_sc as plsc

N = 10000
B = 4
C = 128
E = 320000

NC = 2
NS = 16
CH = 128

NJ = 10240
SROWS = NJ // NS
ZB = 160

CHUNKS_A = 80
E_PAD_A = NC * NS * CHUNKS_A * CH
NGA = CHUNKS_A // 4

CHUNKS_C = 157
E_PAD_C = NS * CHUNKS_C * CH

_mesh = functools.partial(
    plsc.VectorSubcoreMesh, core_axis_name="c", subcore_axis_name="s"
)


def _deg_body(colp, ones, zeros, deg_out, ones_v, ci_v, acc_sh):
  c = lax.axis_index("c")
  s = lax.axis_index("s")
  pltpu.sync_copy(ones, ones_v)
  for j in range(SROWS // ZB):
    pltpu.sync_copy(zeros, acc_sh.at[pl.ds(s * SROWS + j * ZB, ZB)])
  plsc.subcore_barrier()

  base = (c * NS + s) * (CHUNKS_A * CH)

  def grp(g, _):
    pltpu.sync_copy(colp.at[pl.ds(base + g * CH, CH)], ci_v)
    pltpu.sync_copy(ones_v, acc_sh.at[ci_v], add=True)
    return 0

  lax.fori_loop(0, CHUNKS_A, grp, 0)
  plsc.subcore_barrier()
  pltpu.sync_copy(
      acc_sh.at[pl.ds(s * SROWS, SROWS)],
      deg_out.at[pl.ds(c * NJ + s * SROWS, SROWS)],
  )


def _scat_body(rowsp, colp, xs, out, ri_v, ci_v, gbuf, sem, acc_sh):
  c = lax.axis_index("c")
  s = lax.axis_index("s")
  for r in range(2):
    b = 2 * r + c
    pltpu.sync_copy(
        xs.at[pl.ds(b * NJ + s * SROWS, SROWS)],
        acc_sh.at[pl.ds(s * SROWS, SROWS)],
    )
    plsc.subcore_barrier()
    ebase = b * E_PAD_C + s * (CHUNKS_C * CH)
    cbase = s * (CHUNKS_C * CH)

    def body(g, _):
      pltpu.sync_copy(rowsp.at[pl.ds(ebase + g * CH, CH)], ri_v)
      pltpu.async_copy(xs.at[ri_v], gbuf, sem).wait()
      pltpu.sync_copy(colp.at[pl.ds(cbase + g * CH, CH)], ci_v)
      pltpu.sync_copy(gbuf, acc_sh.at[ci_v], add=True)
      return 0

    lax.fori_loop(0, CHUNKS_C, body, 0)
    plsc.subcore_barrier()
    pltpu.sync_copy(
        acc_sh.at[pl.ds(s * SROWS, SROWS)],
        out.at[pl.ds(b * NJ + s * SROWS, SROWS)],
    )
    plsc.subcore_barrier()


def _dis_block(d0_ref, d1_ref):
  deg = d0_ref[:, 0:1] + d1_ref[:, 0:1] + 1.0
  return lax.rsqrt(deg)


def _mm_body(x_ref, w_ref, d0_ref, d1_ref, o_ref):
  xw = jnp.dot(x_ref[...], w_ref[...], preferred_element_type=jnp.float32)
  o_ref[...] = xw * _dis_block(d0_ref, d1_ref)


def _fin_body(a_ref, d0_ref, d1_ref, b_ref, o_ref):
  o_ref[...] = a_ref[...] * _dis_block(d0_ref, d1_ref) + b_ref[...]


def kernel(x, edge_index, W, b):
  row = edge_index[0].astype(jnp.int32)
  col = edge_index[1].astype(jnp.int32)

  colp_a = jnp.concatenate([col, jnp.full((E_PAD_A - E,), N, jnp.int32)])
  colp_c = jnp.concatenate([col, jnp.full((E_PAD_C - E,), N, jnp.int32)])
  rowp = jnp.concatenate([row, jnp.zeros((E_PAD_C - E,), jnp.int32)])
  rows4 = (rowp[None, :] + (jnp.arange(B, dtype=jnp.int32) * NJ)[:, None])
  rows4 = rows4.reshape(-1)

  xp = jnp.pad(x, ((0, 0), (0, NJ - N), (0, 0))).reshape(B * NJ, C)

  ones = jnp.ones((CH, 128), jnp.float32)
  zeros = jnp.zeros((ZB, 128), jnp.float32)

  deg_part = pl.kernel(
      _deg_body,
      out_type=jax.ShapeDtypeStruct((NC * NJ, 128), jnp.float32),
      mesh=_mesh(),
      scratch_types=[
          pltpu.VMEM((CH, 128), jnp.float32),
          pltpu.VMEM((CH,), jnp.int32),
          pltpu.VMEM_SHARED((NJ, 128), jnp.float32),
      ],
      name="gcn_degree_sc",
  )(colp_a, ones, zeros)
  deg0 = deg_part[:NJ]
  deg1 = deg_part[NJ:]

  grid = (B * NJ // SROWS,)
  dmap = lambda i: (i % NS, 0)
  xs = pl.pallas_call(
      _mm_body,
      grid=grid,
      in_specs=[
          pl.BlockSpec((SROWS, C), lambda i: (i, 0)),
          pl.BlockSpec((C, C), lambda i: (0, 0)),
          pl.BlockSpec((SROWS, 128), dmap),
          pl.BlockSpec((SROWS, 128), dmap),
      ],
      out_specs=pl.BlockSpec((SROWS, C), lambda i: (i, 0)),
      out_shape=jax.ShapeDtypeStruct((B * NJ, C), jnp.float32),
      name="gcn_xw_scale_tc",
  )(xp, W, deg0, deg1)

  acc = pl.kernel(
      _scat_body,
      out_type=jax.ShapeDtypeStruct((B * NJ, C), jnp.float32),
      mesh=_mesh(),
      scratch_types=[
          pltpu.VMEM((CH,), jnp.int32),
          pltpu.VMEM((CH,), jnp.int32),
          pltpu.VMEM((CH, C), jnp.float32),
          pltpu.SemaphoreType.DMA,
          pltpu.VMEM_SHARED((NJ, C), jnp.float32),
      ],
      name="gcn_edge_scatter_sc",
  )(rows4, colp_c, xs)

  out = pl.pallas_call(
      _fin_body,
      grid=grid,
      in_specs=[
          pl.BlockSpec((SROWS, C), lambda i: (i, 0)),
          pl.BlockSpec((SROWS, 128), dmap),
          pl.BlockSpec((SROWS, 128), dmap),
          pl.BlockSpec((1, C), lambda i: (0, 0)),
      ],
      out_specs=pl.BlockSpec((SROWS, C), lambda i: (i, 0)),
      out_shape=jax.ShapeDtypeStruct((B * NJ, C), jnp.float32),
      name="gcn_finalize_tc",
  )(acc, deg0, deg1, b.reshape(1, C))

  return out.reshape(B, NJ, C)[:, :N]
